# Initial kernel scaffold; baseline (speedup 1.0000x reference)
import jax
import jax.numpy as jnp
from jax.experimental import pallas as pl


def kernel(x, edge_index, edge_weight, W, a_src, a_dst):
    raise NotImplementedError("write your pallas kernel here")


# speedup vs baseline: 17.7394x; 17.7394x over previous
import functools

import jax
import jax.numpy as jnp
from jax import lax
from jax.experimental import pallas as pl
from jax.experimental.pallas import tpu as pltpu
from jax.experimental.pallas import tpu_sc as plsc

N = 10000
E = 320000
D = 128
H = 4
SLOPE = 0.2

NC = 2
NS = 16
NPAD = 10240
NPT = NPAD // NS
DRAIN = 80

G = 80
MJ = 25
BM = MJ * G
EPT = E // NS
NMAC = EPT // BM
ER = E // G
RPT = ER // NS

BN = 1024

_SC_PARAMS = pltpu.CompilerParams(needs_layout_passes=False,
                                  use_tc_tiling_on_sc=False)


def _tc_body(x_ref, w_ref, asr_ref, adr_ref, h_ref, as_ref, ad_ref):
    xb = x_ref[...]
    for hh in range(H):
        hb = jnp.dot(xb, w_ref[hh], preferred_element_type=jnp.float32)
        h_ref[hh] = hb
        asv = asr_ref[hh, :, 0][None, :]
        adv = adr_ref[hh, :, 0][None, :]
        as_ref[hh, :] = jnp.sum(hb * asv, axis=1)
        ad_ref[hh, :] = jnp.sum(hb * adv, axis=1)


def _project(x, W, a_src, a_dst):
    grid = (pl.cdiv(N, BN),)
    return pl.pallas_call(
        _tc_body,
        grid=grid,
        in_specs=[
            pl.BlockSpec((BN, D), lambda i: (i, 0)),
            pl.BlockSpec((H, D, D), lambda i: (0, 0, 0)),
            pl.BlockSpec((H, D, 1), lambda i: (0, 0, 0)),
            pl.BlockSpec((H, D, 1), lambda i: (0, 0, 0)),
        ],
        out_specs=[
            pl.BlockSpec((H, BN, D), lambda i: (0, i, 0)),
            pl.BlockSpec((H, BN), lambda i: (0, i)),
            pl.BlockSpec((H, BN), lambda i: (0, i)),
        ],
        out_shape=[
            jax.ShapeDtypeStruct((H, N, D), jnp.float32),
            jax.ShapeDtypeStruct((H, N), jnp.float32),
            jax.ShapeDtypeStruct((H, N), jnp.float32),
        ],
    )(x, W, a_src, a_dst)


def _phase_a_body(asf, adf, srcs, dsts, ews, e_out, rcp_out,
                  denom, as_t, ad_t, src_m, dst_m, w_m, e_m, didx):
    c = lax.axis_index("c")
    s = lax.axis_index("s")
    zero16 = jnp.zeros((16,), jnp.float32)

    for hp in range(2):
        head = 2 * c + hp
        pltpu.sync_copy(asf.at[pl.ds(head * N, N)], as_t.at[pl.ds(hp * N, N)])
        pltpu.sync_copy(adf.at[pl.ds(head * N, N)], ad_t.at[pl.ds(hp * N, N)])

    def _z16(i, _):
        w_m[0, pl.ds(i * 16, 16)] = zero16
        return 0
    lax.fori_loop(0, G // 16, _z16, 0)
    for hp in range(2):
        def _zc(k, _):
            pltpu.sync_copy(w_m.at[0],
                            denom.at[pl.ds(hp * NPAD + s * NPT + k * DRAIN, DRAIN)])
            return 0
        lax.fori_loop(0, NPT // DRAIN, _zc, 0)
    plsc.subcore_barrier()

    def _mac(m, _):
        rb = s * RPT + m * MJ
        pltpu.sync_copy(srcs.at[pl.ds(rb, MJ)], src_m)
        pltpu.sync_copy(dsts.at[pl.ds(rb, MJ)], dst_m)
        pltpu.sync_copy(ews.at[pl.ds(rb, MJ)], w_m)
        for hp in range(2):
            head = 2 * c + hp

            def _grp(j, _):
                def _e16(i, _):
                    s16 = src_m[j, pl.ds(i * 16, 16)]
                    d16 = dst_m[j, pl.ds(i * 16, 16)]
                    l = (plsc.load_gather(as_t, [s16 + hp * N])
                         + plsc.load_gather(ad_t, [d16 + hp * N]))
                    l = jnp.where(l >= 0.0, l, l * SLOPE) * w_m[j, pl.ds(i * 16, 16)]
                    e_m[j, pl.ds(i * 16, 16)] = jnp.exp(l)
                    didx[j, pl.ds(i * 16, 16)] = d16 + hp * NPAD
                    return 0
                lax.fori_loop(0, G // 16, _e16, 0)
                pltpu.sync_copy(e_m.at[j], denom.at[didx.at[j]], add=True)
                return 0
            lax.fori_loop(0, MJ, _grp, 0)
            pltpu.sync_copy(e_m, e_out.at[pl.ds(head * ER + rb, MJ)])
        return 0
    lax.fori_loop(0, NMAC, _mac, 0)
    plsc.subcore_barrier()

    for hp in range(2):
        head = 2 * c + hp

        def _dr(k, _):
            nb = s * NPT + k * DRAIN
            pltpu.sync_copy(denom.at[pl.ds(hp * NPAD + nb, DRAIN)], w_m.at[0])

            def _r16(i, _):
                d16 = w_m[0, pl.ds(i * 16, 16)]
                e_m[0, pl.ds(i * 16, 16)] = 1.0 / (d16 + 1e-10)
                return 0
            lax.fori_loop(0, DRAIN // 16, _r16, 0)
            pltpu.sync_copy(e_m.at[0], rcp_out.at[pl.ds(head * NPAD + nb, DRAIN)])
            return 0
        lax.fori_loop(0, NPT // DRAIN, _dr, 0)


def _phase_b_body(rcpf, srcs, dsts, ef, h2, out,
                  acc, rcp_t, src_m, dst_m, e_m, idx_m, gbuf, gsem):
    c = lax.axis_index("c")
    s = lax.axis_index("s")
    zero16 = jnp.zeros((16,), jnp.float32)

    for hp in range(2):
        head = 2 * c + hp
        hoff = head * N
        pltpu.sync_copy(rcpf.at[pl.ds(head * NPAD, NPAD)], rcp_t)

        def _zrow(r, _):
            for j in range(D // 16):
                gbuf[r, pl.ds(16 * j, 16)] = zero16
            return 0
        lax.fori_loop(0, DRAIN, _zrow, 0)

        def _zc(k, _):
            pltpu.sync_copy(gbuf, acc.at[pl.ds(s * NPT + k * DRAIN, DRAIN)])
            return 0
        lax.fori_loop(0, NPT // DRAIN, _zc, 0)
        plsc.subcore_barrier()

        def _mac(m, _):
            rb = s * RPT + m * MJ
            pltpu.sync_copy(srcs.at[pl.ds(rb, MJ)], src_m)
            pltpu.sync_copy(dsts.at[pl.ds(rb, MJ)], dst_m)
            pltpu.sync_copy(ef.at[pl.ds(head * ER + rb, MJ)], e_m)

            def _grp(j, _):
                def _a16(i, _):
                    d16 = dst_m[j, pl.ds(i * 16, 16)]
                    e_m[j, pl.ds(i * 16, 16)] = (e_m[j, pl.ds(i * 16, 16)]
                                                 * plsc.load_gather(rcp_t, [d16]))
                    idx_m[j, pl.ds(i * 16, 16)] = src_m[j, pl.ds(i * 16, 16)] + hoff
                    return 0
                lax.fori_loop(0, G // 16, _a16, 0)
                pltpu.async_copy(h2.at[idx_m.at[j]], gbuf, gsem).wait()

                def _row(r, _):
                    a16 = plsc.load_gather(
                        e_m, [jnp.full((16,), j, jnp.int32),
                              jnp.full((16,), r, jnp.int32)])
                    for jj in range(D // 16):
                        gbuf[r, pl.ds(16 * jj, 16)] = gbuf[r, pl.ds(16 * jj, 16)] * a16
                    return 0
                lax.fori_loop(0, G, _row, 0)

                pltpu.sync_copy(gbuf, acc.at[dst_m.at[j]], add=True)
                return 0
            lax.fori_loop(0, MJ, _grp, 0)
            return 0
        lax.fori_loop(0, NMAC, _mac, 0)
        plsc.subcore_barrier()

        def _drain(k2, _):
            nb = s * NPT + k2 * DRAIN
            pltpu.sync_copy(acc.at[pl.ds(nb, DRAIN)], gbuf)
            pltpu.sync_copy(gbuf, out.at[pl.ds(head * NPAD + nb, DRAIN)])
            return 0
        lax.fori_loop(0, NPT // DRAIN, _drain, 0)


def _sc_phase_a(attn_src, attn_dst, src, dst, ew):
    mesh = plsc.VectorSubcoreMesh(core_axis_name="c", subcore_axis_name="s",
                                  num_cores=NC, num_subcores=NS)
    k = functools.partial(
        pl.kernel,
        out_type=[
            jax.ShapeDtypeStruct((H * ER, G), jnp.float32),
            jax.ShapeDtypeStruct((H * NPAD,), jnp.float32),
        ],
        mesh=mesh,
        scratch_types=[
            pltpu.VMEM_SHARED((2 * NPAD,), jnp.float32),
            pltpu.VMEM((2 * N,), jnp.float32),
            pltpu.VMEM((2 * N,), jnp.float32),
            pltpu.VMEM((MJ, G), jnp.int32),
            pltpu.VMEM((MJ, G), jnp.int32),
            pltpu.VMEM((MJ, G), jnp.float32),
            pltpu.VMEM((MJ, G), jnp.float32),
            pltpu.VMEM((MJ, G), jnp.int32),
        ],
        compiler_params=_SC_PARAMS,
    )(_phase_a_body)
    return k(attn_src, attn_dst, src, dst, ew)


def _sc_phase_b(rcp, src, dst, e_exp, h2):
    mesh = plsc.VectorSubcoreMesh(core_axis_name="c", subcore_axis_name="s",
                                  num_cores=NC, num_subcores=NS)
    k = functools.partial(
        pl.kernel,
        out_type=jax.ShapeDtypeStruct((H * NPAD, D), jnp.float32),
        mesh=mesh,
        scratch_types=[
            pltpu.VMEM_SHARED((NPAD, D), jnp.float32),
            pltpu.VMEM((NPAD,), jnp.float32),
            pltpu.VMEM((MJ, G), jnp.int32),
            pltpu.VMEM((MJ, G), jnp.int32),
            pltpu.VMEM((MJ, G), jnp.float32),
            pltpu.VMEM((MJ, G), jnp.int32),
            pltpu.VMEM((G, D), jnp.float32),
            pltpu.SemaphoreType.DMA,
        ],
        compiler_params=_SC_PARAMS,
    )(_phase_b_body)
    return k(rcp, src, dst, e_exp, h2)


def kernel(x, edge_index, edge_weight, W, a_src, a_dst):
    h, attn_src, attn_dst = _project(x, W, a_src, a_dst)
    src = edge_index[0].reshape(ER, G)
    dst = edge_index[1].reshape(ER, G)
    ew = edge_weight.reshape(ER, G)
    e_exp, rcp = _sc_phase_a(attn_src.reshape(-1), attn_dst.reshape(-1),
                             src, dst, ew)
    h2 = h.reshape(H * N, D)
    out = _sc_phase_b(rcp, src, dst, e_exp, h2)
    out = out.reshape(H, NPAD, D)[:, :N]
    return jnp.transpose(out, (1, 0, 2)).reshape(N, H * D)


# speedup vs baseline: 26.9179x; 1.5174x over previous
import functools

import jax
import jax.numpy as jnp
from jax import lax
from jax.experimental import pallas as pl
from jax.experimental.pallas import tpu as pltpu
from jax.experimental.pallas import tpu_sc as plsc

N = 10000
E = 320000
D = 128
H = 4
SLOPE = 0.2

NC = 2
NS = 16
NPAD = 10240
NPT = NPAD // NS
DRAIN = 80

G = 80
MJ = 25
BM = MJ * G
EPT = E // NS
NMAC = EPT // BM
ER = E // G
RPT = ER // NS

BN = 1024

_SC_PARAMS = pltpu.CompilerParams(needs_layout_passes=False,
                                  use_tc_tiling_on_sc=False)


def _tc_body(x_ref, w_ref, asr_ref, adr_ref, h_ref, as_ref, ad_ref):
    xb = x_ref[...]
    for hh in range(H):
        hb = jnp.dot(xb, w_ref[hh], preferred_element_type=jnp.float32)
        h_ref[hh] = hb
        asv = asr_ref[hh, :, 0][None, :]
        adv = adr_ref[hh, :, 0][None, :]
        as_ref[hh, :] = jnp.sum(hb * asv, axis=1)
        ad_ref[hh, :] = jnp.sum(hb * adv, axis=1)


def _project(x, W, a_src, a_dst):
    grid = (pl.cdiv(N, BN),)
    return pl.pallas_call(
        _tc_body,
        grid=grid,
        in_specs=[
            pl.BlockSpec((BN, D), lambda i: (i, 0)),
            pl.BlockSpec((H, D, D), lambda i: (0, 0, 0)),
            pl.BlockSpec((H, D, 1), lambda i: (0, 0, 0)),
            pl.BlockSpec((H, D, 1), lambda i: (0, 0, 0)),
        ],
        out_specs=[
            pl.BlockSpec((H, BN, D), lambda i: (0, i, 0)),
            pl.BlockSpec((H, BN), lambda i: (0, i)),
            pl.BlockSpec((H, BN), lambda i: (0, i)),
        ],
        out_shape=[
            jax.ShapeDtypeStruct((H, N, D), jnp.float32),
            jax.ShapeDtypeStruct((H, N), jnp.float32),
            jax.ShapeDtypeStruct((H, N), jnp.float32),
        ],
    )(x, W, a_src, a_dst)


def _phase_a_body(asf, adf, srcs, dsts, ews, e_out, rcp_out,
                  denom, as_t, ad_t, src_m, dst_m, w_m, e_m, didx):
    c = lax.axis_index("c")
    s = lax.axis_index("s")
    zero16 = jnp.zeros((16,), jnp.float32)

    for hp in range(2):
        head = 2 * c + hp
        pltpu.sync_copy(asf.at[pl.ds(head * N, N)], as_t.at[pl.ds(hp * N, N)])
        pltpu.sync_copy(adf.at[pl.ds(head * N, N)], ad_t.at[pl.ds(hp * N, N)])

    def _z16(i, _):
        w_m[0, pl.ds(i * 16, 16)] = zero16
        return 0
    lax.fori_loop(0, G // 16, _z16, 0)
    for hp in range(2):
        def _zc(k, _):
            pltpu.sync_copy(w_m.at[0],
                            denom.at[pl.ds(hp * NPAD + s * NPT + k * DRAIN, DRAIN)])
            return 0
        lax.fori_loop(0, NPT // DRAIN, _zc, 0)
    plsc.subcore_barrier()

    def _mac(m, _):
        rb = s * RPT + m * MJ
        pltpu.sync_copy(srcs.at[pl.ds(rb, MJ)], src_m)
        pltpu.sync_copy(dsts.at[pl.ds(rb, MJ)], dst_m)
        pltpu.sync_copy(ews.at[pl.ds(rb, MJ)], w_m)
        for hp in range(2):
            head = 2 * c + hp

            def _grp(j, _):
                def _e16(i, _):
                    s16 = src_m[j, pl.ds(i * 16, 16)]
                    d16 = dst_m[j, pl.ds(i * 16, 16)]
                    l = (plsc.load_gather(as_t, [s16 + hp * N])
                         + plsc.load_gather(ad_t, [d16 + hp * N]))
                    l = jnp.where(l >= 0.0, l, l * SLOPE) * w_m[j, pl.ds(i * 16, 16)]
                    e_m[j, pl.ds(i * 16, 16)] = jnp.exp(l)
                    didx[j, pl.ds(i * 16, 16)] = d16 + hp * NPAD
                    return 0
                lax.fori_loop(0, G // 16, _e16, 0)
                pltpu.sync_copy(e_m.at[j], denom.at[didx.at[j]], add=True)
                return 0
            lax.fori_loop(0, MJ, _grp, 0)
            pltpu.sync_copy(e_m, e_out.at[pl.ds(head * ER + rb, MJ)])
        return 0
    lax.fori_loop(0, NMAC, _mac, 0)
    plsc.subcore_barrier()

    for hp in range(2):
        head = 2 * c + hp

        def _dr(k, _):
            nb = s * NPT + k * DRAIN
            pltpu.sync_copy(denom.at[pl.ds(hp * NPAD + nb, DRAIN)], w_m.at[0])

            def _r16(i, _):
                d16 = w_m[0, pl.ds(i * 16, 16)]
                e_m[0, pl.ds(i * 16, 16)] = 1.0 / (d16 + 1e-10)
                return 0
            lax.fori_loop(0, DRAIN // 16, _r16, 0)
            pltpu.sync_copy(e_m.at[0], rcp_out.at[pl.ds(head * NPAD + nb, DRAIN)])
            return 0
        lax.fori_loop(0, NPT // DRAIN, _dr, 0)


def _phase_b_body(rcpf, srcs, dsts, ef, h2, out,
                  acc, rcp_t, src_m, dst_m, e_m, idx_m, gbuf, gbuf2, gsem, gsem2):
    c = lax.axis_index("c")
    s = lax.axis_index("s")
    zero16 = jnp.zeros((16,), jnp.float32)

    for hp in range(2):
        head = 2 * c + hp
        hoff = head * N
        pltpu.sync_copy(rcpf.at[pl.ds(head * NPAD, NPAD)], rcp_t)

        def _zrow(r, _):
            for j in range(D // 16):
                gbuf[r, pl.ds(16 * j, 16)] = zero16
            return 0
        lax.fori_loop(0, DRAIN, _zrow, 0)

        def _zc(k, _):
            pltpu.sync_copy(gbuf, acc.at[pl.ds(s * NPT + k * DRAIN, DRAIN)])
            return 0
        lax.fori_loop(0, NPT // DRAIN, _zc, 0)
        plsc.subcore_barrier()

        def _prep(j):
            def _a16(i, _):
                d16 = dst_m[j, pl.ds(i * 16, 16)]
                e_m[j, pl.ds(i * 16, 16)] = (e_m[j, pl.ds(i * 16, 16)]
                                             * plsc.load_gather(rcp_t, [d16]))
                idx_m[j, pl.ds(i * 16, 16)] = src_m[j, pl.ds(i * 16, 16)] + hoff
                return 0
            lax.fori_loop(0, G // 16, _a16, 0)

        def _consume(j, gb):
            def _row(r, _):
                a16 = plsc.load_gather(
                    e_m, [jnp.full((16,), j, jnp.int32),
                          jnp.full((16,), r, jnp.int32)])
                for jj in range(D // 16):
                    gb[r, pl.ds(16 * jj, 16)] = gb[r, pl.ds(16 * jj, 16)] * a16
                return 0
            lax.fori_loop(0, G, _row, 0)
            pltpu.sync_copy(gb, acc.at[dst_m.at[j]], add=True)

        def _mac(m, _):
            rb = s * RPT + m * MJ
            pltpu.sync_copy(srcs.at[pl.ds(rb, MJ)], src_m)
            pltpu.sync_copy(dsts.at[pl.ds(rb, MJ)], dst_m)
            pltpu.sync_copy(ef.at[pl.ds(head * ER + rb, MJ)], e_m)

            _prep(0)
            pltpu.make_async_copy(h2.at[idx_m.at[0]], gbuf, gsem).start()

            def _pair(k, _):
                j0 = 2 * k
                j1 = 2 * k + 1
                j2 = 2 * k + 2
                _prep(j1)
                cpb = pltpu.make_async_copy(h2.at[idx_m.at[j1]], gbuf2, gsem2)
                cpb.start()
                pltpu.make_async_copy(h2.at[idx_m.at[j0]], gbuf, gsem).wait()
                _consume(j0, gbuf)
                _prep(j2)
                cpa = pltpu.make_async_copy(h2.at[idx_m.at[j2]], gbuf, gsem)
                cpa.start()
                pltpu.make_async_copy(h2.at[idx_m.at[j1]], gbuf2, gsem2).wait()
                _consume(j1, gbuf2)
                return 0
            lax.fori_loop(0, (MJ - 1) // 2, _pair, 0)
            pltpu.make_async_copy(h2.at[idx_m.at[MJ - 1]], gbuf, gsem).wait()
            _consume(MJ - 1, gbuf)
            return 0
        lax.fori_loop(0, NMAC, _mac, 0)
        plsc.subcore_barrier()

        def _drain(k2, _):
            nb = s * NPT + k2 * DRAIN
            pltpu.sync_copy(acc.at[pl.ds(nb, DRAIN)], gbuf)
            pltpu.sync_copy(gbuf, out.at[pl.ds(head * NPAD + nb, DRAIN)])
            return 0
        lax.fori_loop(0, NPT // DRAIN, _drain, 0)


def _sc_phase_a(attn_src, attn_dst, src, dst, ew):
    mesh = plsc.VectorSubcoreMesh(core_axis_name="c", subcore_axis_name="s",
                                  num_cores=NC, num_subcores=NS)
    k = functools.partial(
        pl.kernel,
        out_type=[
            jax.ShapeDtypeStruct((H * ER, G), jnp.float32),
            jax.ShapeDtypeStruct((H * NPAD,), jnp.float32),
        ],
        mesh=mesh,
        scratch_types=[
            pltpu.VMEM_SHARED((2 * NPAD,), jnp.float32),
            pltpu.VMEM((2 * N,), jnp.float32),
            pltpu.VMEM((2 * N,), jnp.float32),
            pltpu.VMEM((MJ, G), jnp.int32),
            pltpu.VMEM((MJ, G), jnp.int32),
            pltpu.VMEM((MJ, G), jnp.float32),
            pltpu.VMEM((MJ, G), jnp.float32),
            pltpu.VMEM((MJ, G), jnp.int32),
        ],
        compiler_params=_SC_PARAMS,
    )(_phase_a_body)
    return k(attn_src, attn_dst, src, dst, ew)


def _sc_phase_b(rcp, src, dst, e_exp, h2):
    mesh = plsc.VectorSubcoreMesh(core_axis_name="c", subcore_axis_name="s",
                                  num_cores=NC, num_subcores=NS)
    k = functools.partial(
        pl.kernel,
        out_type=jax.ShapeDtypeStruct((H * NPAD, D), jnp.float32),
        mesh=mesh,
        scratch_types=[
            pltpu.VMEM_SHARED((NPAD, D), jnp.float32),
            pltpu.VMEM((NPAD,), jnp.float32),
            pltpu.VMEM((MJ, G), jnp.int32),
            pltpu.VMEM((MJ, G), jnp.int32),
            pltpu.VMEM((MJ, G), jnp.float32),
            pltpu.VMEM((MJ, G), jnp.int32),
            pltpu.VMEM((G, D), jnp.float32),
            pltpu.VMEM((G, D), jnp.float32),
            pltpu.SemaphoreType.DMA,
            pltpu.SemaphoreType.DMA,
        ],
        compiler_params=_SC_PARAMS,
    )(_phase_b_body)
    return k(rcp, src, dst, e_exp, h2)


def kernel(x, edge_index, edge_weight, W, a_src, a_dst):
    h, attn_src, attn_dst = _project(x, W, a_src, a_dst)
    src = edge_index[0].reshape(ER, G)
    dst = edge_index[1].reshape(ER, G)
    ew = edge_weight.reshape(ER, G)
    e_exp, rcp = _sc_phase_a(attn_src.reshape(-1), attn_dst.reshape(-1),
                             src, dst, ew)
    h2 = h.reshape(H * N, D)
    out = _sc_phase_b(rcp, src, dst, e_exp, h2)
    out = out.reshape(H, NPAD, D)[:, :N]
    return jnp.transpose(out, (1, 0, 2)).reshape(N, H * D)
